# baseline (speedup 1.0000x reference)
import functools

import jax
import jax.numpy as jnp
from jax import lax
from jax.experimental import pallas as pl
from jax.experimental.pallas import tpu as pltpu
from jax.experimental.pallas import tpu_sc as plsc

N = 10000
E = 160000
T = 320000
H = 256
CENTERS = 80
TRIP = 40
NG = 64

_NC = 2
_NS = 16
_NW = _NC * _NS
_B = 40


def _silu(x):
    return x * jax.nn.sigmoid(x)


def _bn(x):
    m = jnp.mean(x, axis=0)
    v = jnp.var(x, axis=0)
    return (x - m) / jnp.sqrt(v + 1e-5)


def _rbf(d, vmin, vmax, bins):
    centers = jnp.linspace(vmin, vmax, bins)
    gamma = 1.0 / ((vmax - vmin) / (bins - 1))
    return jnp.exp(-gamma * (d - centers) ** 2)


def _mm_body(x_ref, w_ref, b_ref, o_ref):
    o_ref[...] = (
        jnp.dot(x_ref[...], w_ref[...], preferred_element_type=jnp.float32)
        + b_ref[...]
    )


def _mm(x, W, b, bm=1000):
    R, K = x.shape
    O = W.shape[1]
    return pl.pallas_call(
        _mm_body,
        grid=(R // bm,),
        in_specs=[
            pl.BlockSpec((bm, K), lambda r: (r, 0)),
            pl.BlockSpec((K, O), lambda r: (0, 0)),
            pl.BlockSpec((1, O), lambda r: (0, 0)),
        ],
        out_specs=pl.BlockSpec((bm, O), lambda r: (r, 0)),
        out_shape=jax.ShapeDtypeStruct((R, O), jnp.float32),
    )(x, W, b.reshape(1, -1))


@functools.lru_cache(maxsize=None)
def _make_gate(nt):
    per_w = nt // _NW
    nblk = per_w // _B
    assert per_w * _NW == nt and nblk * _B == per_w

    mesh = plsc.VectorSubcoreMesh(core_axis_name="c", subcore_axis_name="s")

    @functools.partial(
        pl.kernel,
        mesh=mesh,
        out_type=[
            jax.ShapeDtypeStruct((nt, H), jnp.float32),
            jax.ShapeDtypeStruct((nt, H), jnp.float32),
            jax.ShapeDtypeStruct((nt, H), jnp.float32),
        ],
        scratch_types=[
            pltpu.VMEM((_B,), jnp.int32),
            pltpu.VMEM((_B,), jnp.int32),
            pltpu.VMEM((_B, H), jnp.float32),
            pltpu.VMEM((_B, H), jnp.float32),
            pltpu.VMEM((_B, H), jnp.float32),
            pltpu.VMEM((_B, H), jnp.float32),
            pltpu.SemaphoreType.DMA,
        ],
    )
    def gate(i_hbm, j_hbm, es_hbm, ed_hbm, bh_hbm, eg_hbm,
             yg_hbm, sg_hbm, m_hbm,
             ii_v, jj_v, es_v, ed_v, bh_v, eg_v, sem):
        w = lax.axis_index("s") * _NC + lax.axis_index("c")
        base0 = w * per_w

        def blk(g, carry):
            base = base0 + g * _B
            pltpu.sync_copy(i_hbm.at[pl.ds(base, _B)], ii_v)
            pltpu.sync_copy(j_hbm.at[pl.ds(base, _B)], jj_v)
            c1 = pltpu.async_copy(es_hbm.at[ii_v], es_v, sem)
            c2 = pltpu.async_copy(ed_hbm.at[jj_v], ed_v, sem)
            c3 = pltpu.async_copy(bh_hbm.at[jj_v], bh_v, sem)
            c4 = pltpu.async_copy(eg_hbm.at[pl.ds(base, _B)], eg_v, sem)
            c1.wait()
            c2.wait()
            c3.wait()
            c4.wait()

            def row(r, cr):
                for cc in range(H // 16):
                    sl = pl.ds(cc * 16, 16)
                    yg = es_v[r, sl] + ed_v[r, sl] + eg_v[r, sl]
                    sig = 1.0 / (1.0 + jnp.exp(-yg))
                    m = bh_v[r, sl] * sig
                    es_v[r, sl] = yg
                    ed_v[r, sl] = sig
                    bh_v[r, sl] = m
                return cr

            lax.fori_loop(0, _B, row, 0, unroll=False)
            pltpu.sync_copy(es_v, yg_hbm.at[pl.ds(base, _B)])
            pltpu.sync_copy(ed_v, sg_hbm.at[pl.ds(base, _B)])
            pltpu.sync_copy(bh_v, m_hbm.at[pl.ds(base, _B)])
            return carry

        lax.fori_loop(0, nblk, blk, 0, unroll=False)

    return gate


def _egc(node, edge, i, j, p, n_seg, gate):
    es = _mm(node, p['sgW'], p['sgb'])
    ed = _mm(node, p['dgW'], p['dgb'])
    bh = _mm(node, p['duW'], p['dub'])
    su = _mm(node, p['suW'], p['sub'])
    eg = _mm(edge, p['egW'], p['egb'])
    yg, sg, m = gate(i, j, es, ed, bh, eg)
    ssh = jax.ops.segment_sum(m, i, num_segments=n_seg)
    ss = jax.ops.segment_sum(sg, i, num_segments=n_seg)
    h = ssh / (ss + 1e-6)
    xq = _silu(_bn(su + h))
    yq = _silu(_bn(yg))
    return node + xq, edge + yq


def kernel(x, edge_index, edge_index_triplets, dist, angle, batch, params):
    it = edge_index_triplets[0]
    jt = edge_index_triplets[1]
    ie = edge_index[0]
    je = edge_index[1]

    xh = _silu(_bn(x @ params['atom']['W'] + params['atom']['b']))
    y = _rbf(dist, 0.0, 8.0, CENTERS)
    y = _silu(_bn(_mm(y, params['edge1']['W'], params['edge1']['b'])))
    y = _silu(_bn(_mm(y, params['edge2']['W'], params['edge2']['b'])))
    z = _rbf(angle, -1.0, 1.0, TRIP)
    z = _silu(_bn(_mm(z, params['ang1']['W'], params['ang1']['b'])))
    z = _silu(_bn(_mm(z, params['ang2']['W'], params['ang2']['b'])))
    for lp in params['alignn']:
        m, z = _egc(y, z, it, jt, lp['edge'], E, _make_gate(T))
        xh, y = _egc(xh, m, ie, je, lp['node'], N, _make_gate(E))
    for gp in params['gcn']:
        xh, y = _egc(xh, y, ie, je, gp, N, _make_gate(E))
    sums = jax.ops.segment_sum(xh, batch, num_segments=NG)
    cnt = jax.ops.segment_sum(jnp.ones((N, 1), jnp.float32), batch, num_segments=NG)
    h = sums / jnp.maximum(cnt, 1.0)
    return h @ params['out']['W'] + params['out']['b']


# speedup vs baseline: 1.0341x; 1.0341x over previous
import functools

import jax
import jax.numpy as jnp
from jax import lax
from jax.experimental import pallas as pl
from jax.experimental.pallas import tpu as pltpu
from jax.experimental.pallas import tpu_sc as plsc

N = 10000
E = 160000
T = 320000
H = 256
CENTERS = 80
TRIP = 40
NG = 64

_NC = 2
_NS = 16
_NW = _NC * _NS
_B = 40


def _silu(x):
    return x * jax.nn.sigmoid(x)


def _bn(x):
    m = jnp.mean(x, axis=0)
    v = jnp.var(x, axis=0)
    return (x - m) / jnp.sqrt(v + 1e-5)


def _rbf(d, vmin, vmax, bins):
    centers = jnp.linspace(vmin, vmax, bins)
    gamma = 1.0 / ((vmax - vmin) / (bins - 1))
    return jnp.exp(-gamma * (d - centers) ** 2)


def _mm_body(x_ref, w_ref, b_ref, o_ref):
    o_ref[...] = (
        jnp.dot(x_ref[...], w_ref[...], preferred_element_type=jnp.float32)
        + b_ref[...]
    )


def _mm(x, W, b, bm=1000):
    R, K = x.shape
    O = W.shape[1]
    return pl.pallas_call(
        _mm_body,
        grid=(R // bm,),
        in_specs=[
            pl.BlockSpec((bm, K), lambda r: (r, 0)),
            pl.BlockSpec((K, O), lambda r: (0, 0)),
            pl.BlockSpec((1, O), lambda r: (0, 0)),
        ],
        out_specs=pl.BlockSpec((bm, O), lambda r: (r, 0)),
        out_shape=jax.ShapeDtypeStruct((R, O), jnp.float32),
    )(x, W, b.reshape(1, -1))


@functools.lru_cache(maxsize=None)
def _make_gate(nt):
    per_w = nt // _NW
    nblk = per_w // _B
    assert per_w * _NW == nt and nblk * _B == per_w

    mesh = plsc.VectorSubcoreMesh(core_axis_name="c", subcore_axis_name="s")

    @functools.partial(
        pl.kernel,
        mesh=mesh,
        out_type=[
            jax.ShapeDtypeStruct((nt, H), jnp.float32),
            jax.ShapeDtypeStruct((nt, H), jnp.float32),
            jax.ShapeDtypeStruct((nt, H), jnp.float32),
        ],
        scratch_types=[
            pltpu.VMEM((_B,), jnp.int32),
            pltpu.VMEM((_B,), jnp.int32),
            pltpu.VMEM((_B, H), jnp.float32),
            pltpu.VMEM((_B, H), jnp.float32),
            pltpu.VMEM((_B, H), jnp.float32),
            pltpu.VMEM((_B, H), jnp.float32),
            pltpu.SemaphoreType.DMA,
        ],
    )
    def gate(i_hbm, j_hbm, es_hbm, ed_hbm, bh_hbm, eg_hbm,
             yg_hbm, sg_hbm, m_hbm,
             ii_v, jj_v, es_v, ed_v, bh_v, eg_v, sem):
        w = lax.axis_index("s") * _NC + lax.axis_index("c")
        base0 = w * per_w

        def blk(g, carry):
            base = base0 + g * _B
            pltpu.sync_copy(i_hbm.at[pl.ds(base, _B)], ii_v)
            pltpu.sync_copy(j_hbm.at[pl.ds(base, _B)], jj_v)
            c1 = pltpu.async_copy(es_hbm.at[ii_v], es_v, sem)
            c2 = pltpu.async_copy(ed_hbm.at[jj_v], ed_v, sem)
            c3 = pltpu.async_copy(bh_hbm.at[jj_v], bh_v, sem)
            c4 = pltpu.async_copy(eg_hbm.at[pl.ds(base, _B)], eg_v, sem)
            c1.wait()
            c2.wait()
            c3.wait()
            c4.wait()

            def row(r, cr):
                for cc in range(H // 16):
                    sl = pl.ds(cc * 16, 16)
                    yg = es_v[r, sl] + ed_v[r, sl] + eg_v[r, sl]
                    sig = 1.0 / (1.0 + jnp.exp(-yg))
                    m = bh_v[r, sl] * sig
                    es_v[r, sl] = yg
                    ed_v[r, sl] = sig
                    bh_v[r, sl] = m
                return cr

            lax.fori_loop(0, _B, row, 0, unroll=False)
            pltpu.sync_copy(es_v, yg_hbm.at[pl.ds(base, _B)])
            pltpu.sync_copy(ed_v, sg_hbm.at[pl.ds(base, _B)])
            pltpu.sync_copy(bh_v, m_hbm.at[pl.ds(base, _B)])
            return carry

        lax.fori_loop(0, nblk, blk, 0, unroll=False)

    return gate


@functools.lru_cache(maxsize=None)
def _make_segsum(nt, nseg, Ct):
    nchunk = nseg // Ct
    assert nchunk * Ct == nseg

    mesh = plsc.VectorSubcoreMesh(core_axis_name="c", subcore_axis_name="s")

    @functools.partial(
        pl.kernel,
        mesh=mesh,
        compiler_params=pltpu.CompilerParams(needs_layout_passes=False),
        out_type=[
            jax.ShapeDtypeStruct((nseg, H), jnp.float32),
            jax.ShapeDtypeStruct((nseg, H), jnp.float32),
        ],
        scratch_types=[
            pltpu.VMEM((nchunk + 17,), jnp.int32),
            pltpu.VMEM((64,), jnp.int32),
            pltpu.VMEM((64,), jnp.int32),
            pltpu.VMEM((64, H), jnp.float32),
            pltpu.VMEM((64, H), jnp.float32),
            pltpu.VMEM((Ct + 1, H), jnp.float32),
            pltpu.VMEM((Ct + 1, H), jnp.float32),
            pltpu.SemaphoreType.DMA,
        ],
    )
    def segsum(is_hbm, perm_hbm, rs_hbm, m_hbm, sg_hbm, ssh_hbm, ss_hbm,
               rs_v, posG, ivb, mrow, srow, acc_m, acc_s, sem):
        w = lax.axis_index("s") * _NC + lax.axis_index("c")
        pltpu.sync_copy(rs_hbm, rs_v)

        def zrow(r, cr):
            for cc in range(H // 16):
                sl0 = pl.ds(cc * 16, 16)
                acc_m[r, sl0] = jnp.zeros((16,), jnp.float32)
                acc_s[r, sl0] = jnp.zeros((16,), jnp.float32)
            return cr

        lax.fori_loop(0, Ct + 1, zrow, 0, unroll=False)

        cntw = (nchunk - w + _NW - 1) // _NW
        col_i = [lax.broadcasted_iota(jnp.int32, (16,), 0) + cc * 16
                 for cc in range(H // 16)]

        def chunk_body(k, carry):
            c = w + _NW * k
            seg_base = c * Ct
            bv = rs_v[pl.ds(c, 16)]
            start = bv[0]
            end = bv[1]
            ga = (start // 8) * 8
            ngr = jnp.maximum((end - ga + 63) // 64, 0)

            def gbody(g, cr2):
                bp = ga + g * 64
                pltpu.sync_copy(perm_hbm.at[pl.ds(bp, 64)], posG)
                pltpu.sync_copy(is_hbm.at[pl.ds(bp, 64)], ivb)
                c1 = pltpu.async_copy(m_hbm.at[posG], mrow, sem)
                c2 = pltpu.async_copy(sg_hbm.at[posG], srow, sem)
                c1.wait()
                c2.wait()
                for q in range(4):
                    sl = pl.ds(q * 16, 16)
                    iv = ivb[sl]
                    pvec = lax.broadcasted_iota(jnp.int32, (16,), 0) + (bp + q * 16)
                    valid = (pvec >= start) & (pvec < end)
                    lv = jnp.where(valid, iv - seg_base, Ct)
                    for rr in range(16):
                        rowi = jnp.zeros((16,), jnp.int32) + lv[rr]
                        for cc in range(H // 16):
                            slc = pl.ds(cc * 16, 16)
                            plsc.addupdate_scatter(
                                acc_m, [rowi, col_i[cc]], mrow[q * 16 + rr, slc])
                            plsc.addupdate_scatter(
                                acc_s, [rowi, col_i[cc]], srow[q * 16 + rr, slc])
                return cr2

            lax.fori_loop(0, ngr, gbody, 0, unroll=False)
            pltpu.sync_copy(acc_m.at[pl.ds(0, Ct)], ssh_hbm.at[pl.ds(seg_base, Ct)])
            pltpu.sync_copy(acc_s.at[pl.ds(0, Ct)], ss_hbm.at[pl.ds(seg_base, Ct)])
            lax.fori_loop(0, Ct, zrow, 0, unroll=False)
            return carry

        lax.fori_loop(0, cntw, chunk_body, 0, unroll=False)

    return segsum


_C_T = 128
_C_E = 128
_N_PAD = 10240


def _seg_meta(i_arr, nseg_pad, C):
    perm = jnp.argsort(i_arr).astype(jnp.int32)
    i_s = i_arr[perm].astype(jnp.int32)
    nchunk = nseg_pad // C
    bounds = (jnp.arange(nchunk + 1, dtype=jnp.int32) * C)
    rs = jnp.searchsorted(i_s, bounds).astype(jnp.int32)
    rs = jnp.concatenate([rs, jnp.zeros((16,), jnp.int32)])
    pad = jnp.zeros((128,), jnp.int32)
    return (jnp.concatenate([i_s, pad]), jnp.concatenate([perm, pad]), rs)


def _egc(node, edge, i, j, p, n_seg, gate, segsum, meta):
    es = _mm(node, p['sgW'], p['sgb'])
    ed = _mm(node, p['dgW'], p['dgb'])
    bh = _mm(node, p['duW'], p['dub'])
    su = _mm(node, p['suW'], p['sub'])
    eg = _mm(edge, p['egW'], p['egb'])
    yg, sg, m = gate(i, j, es, ed, bh, eg)
    i_s, perm, rs = meta
    ssh, ss = segsum(i_s, perm, rs, m, sg)
    h = ssh[:n_seg] / (ss[:n_seg] + 1e-6)
    xq = _silu(_bn(su + h))
    yq = _silu(_bn(yg))
    return node + xq, edge + yq


def kernel(x, edge_index, edge_index_triplets, dist, angle, batch, params):
    it = edge_index_triplets[0]
    jt = edge_index_triplets[1]
    ie = edge_index[0]
    je = edge_index[1]

    xh = _silu(_bn(x @ params['atom']['W'] + params['atom']['b']))
    y = _rbf(dist, 0.0, 8.0, CENTERS)
    y = _silu(_bn(_mm(y, params['edge1']['W'], params['edge1']['b'])))
    y = _silu(_bn(_mm(y, params['edge2']['W'], params['edge2']['b'])))
    z = _rbf(angle, -1.0, 1.0, TRIP)
    z = _silu(_bn(_mm(z, params['ang1']['W'], params['ang1']['b'])))
    z = _silu(_bn(_mm(z, params['ang2']['W'], params['ang2']['b'])))
    meta_t = _seg_meta(it, E, _C_T)
    meta_e = _seg_meta(ie, _N_PAD, _C_E)
    seg_t = _make_segsum(T, E, _C_T)
    seg_e = _make_segsum(E, _N_PAD, _C_E)
    for lp in params['alignn']:
        m, z = _egc(y, z, it, jt, lp['edge'], E, _make_gate(T), seg_t, meta_t)
        xh, y = _egc(xh, m, ie, je, lp['node'], N, _make_gate(E), seg_e, meta_e)
    for gp in params['gcn']:
        xh, y = _egc(xh, y, ie, je, gp, N, _make_gate(E), seg_e, meta_e)
    sums = jax.ops.segment_sum(xh, batch, num_segments=NG)
    cnt = jax.ops.segment_sum(jnp.ones((N, 1), jnp.float32), batch, num_segments=NG)
    h = sums / jnp.maximum(cnt, 1.0)
    return h @ params['out']['W'] + params['out']['b']
